# baseline (speedup 1.0000x reference)
import functools

import jax
import jax.numpy as jnp
from jax import lax
from jax.experimental import pallas as pl
from jax.experimental.pallas import tpu as pltpu
from jax.experimental.pallas import tpu_sc as plsc

DIM = 1024
HEADS = 16
DH = 64
GS = 128
WIN = 2 * GS
NB = 2
N = 4096
NG = N // GS
MG = 128
SCALE = DH ** -0.5


def _sc_gather(table, idx, chunk):
    R, D = table.shape
    (M,) = idx.shape
    info = plsc.get_sparse_core_info()
    nw = info.num_cores * info.num_subcores
    per_w = M // nw
    assert per_w * nw == M and per_w % chunk == 0 and chunk % 8 == 0
    nchunks = per_w // chunk
    mesh = plsc.VectorSubcoreMesh(core_axis_name="c", subcore_axis_name="s")

    @functools.partial(
        pl.kernel,
        mesh=mesh,
        out_type=jax.ShapeDtypeStruct((M, D), table.dtype),
        scratch_types=[
            pltpu.VMEM((per_w,), jnp.int32),
            pltpu.VMEM((chunk, D), table.dtype),
            pltpu.VMEM((chunk, D), table.dtype),
            pltpu.SemaphoreType.DMA,
            pltpu.SemaphoreType.DMA,
        ],
    )
    def gk(table_hbm, idx_hbm, out_hbm, idx_v, rows_a, rows_b, sem_a, sem_b):
        wid = lax.axis_index("s") * info.num_cores + lax.axis_index("c")
        base = wid * per_w
        pltpu.sync_copy(idx_hbm.at[pl.ds(base, per_w)], idx_v)
        bufs = (rows_a, rows_b)
        sems = (sem_a, sem_b)
        copies = [
            pltpu.async_copy(
                table_hbm.at[idx_v.at[pl.ds(c * chunk, chunk)]],
                bufs[c % 2],
                sems[c % 2],
            )
            for c in range(min(2, nchunks))
        ]
        for c in range(nchunks):
            copies[c].wait()
            pltpu.sync_copy(bufs[c % 2], out_hbm.at[pl.ds(base + c * chunk, chunk)])
            if c + 2 < nchunks:
                copies.append(
                    pltpu.async_copy(
                        table_hbm.at[idx_v.at[pl.ds((c + 2) * chunk, chunk)]],
                        bufs[c % 2],
                        sems[c % 2],
                    )
                )

    return gk(table, idx)


def _proj_qkv(xg, wq, wk, wv):
    nb = xg.shape[1] // GS
    dn = (((1,), (1,)), ((), ()))
    f32 = jnp.float32
    bf = jnp.bfloat16

    def body(x_ref, wq_ref, wk_ref, wv_ref, q_ref, k_ref, v_ref):
        x = x_ref[0].astype(bf)
        q_ref[0] = lax.dot_general(x, wq_ref[...], dn,
                                   preferred_element_type=f32).astype(bf)
        k_ref[0] = lax.dot_general(x, wk_ref[...], dn,
                                   preferred_element_type=f32).astype(bf)
        v_ref[0] = lax.dot_general(x, wv_ref[...], dn,
                                   preferred_element_type=f32).astype(bf)

    return pl.pallas_call(
        body,
        grid=(NB, nb),
        in_specs=[
            pl.BlockSpec((1, GS, DIM), lambda b, g: (b, g, 0)),
            pl.BlockSpec((DIM, DIM), lambda b, g: (0, 0)),
            pl.BlockSpec((DIM, DIM), lambda b, g: (0, 0)),
            pl.BlockSpec((DIM, DIM), lambda b, g: (0, 0)),
        ],
        out_specs=[
            pl.BlockSpec((1, GS, DIM), lambda b, g: (b, g, 0)),
            pl.BlockSpec((1, GS, DIM), lambda b, g: (b, g, 0)),
            pl.BlockSpec((1, GS, DIM), lambda b, g: (b, g, 0)),
        ],
        out_shape=[
            jax.ShapeDtypeStruct((NB, nb * GS, DIM), jnp.bfloat16),
            jax.ShapeDtypeStruct((NB, nb * GS, DIM), jnp.bfloat16),
            jax.ShapeDtypeStruct((NB, nb * GS, DIM), jnp.bfloat16),
        ],
    )(xg, wq, wk, wv)


def _attn(qbuf, kbuf, vbuf, k_global, v_global, wp):
    c = SCALE * 1.4426950408889634
    bf = jnp.bfloat16
    f32 = jnp.float32
    dn = (((1,), (1,)), ((), ()))

    def body(q_ref, ka_ref, kb_ref, va_ref, vb_ref, kg_ref, vg_ref,
             wp_ref, ones_ref, probs_ref, y_ref, acc_ref):
        ones = ones_ref[...]

        def head_logits(h):
            sl = slice(h * DH, (h + 1) * DH)
            qh = q_ref[0][:, sl]
            la = lax.dot_general(qh, ka_ref[0][:, sl], dn,
                                 preferred_element_type=f32)
            lb = lax.dot_general(qh, kb_ref[0][:, sl], dn,
                                 preferred_element_type=f32)
            gl = lax.dot_general(qh, kg_ref[h], dn,
                                 preferred_element_type=f32)
            return la, lb, gl

        def head_post(h, la, lb, gl):
            sl = slice(h * DH, (h + 1) * DH)
            ea = jnp.exp2(la * c)
            eb = jnp.exp2(lb * c)
            eab = (ea + eb).astype(bf)
            s = jnp.dot(eab, ones, preferred_element_type=f32)
            r = 1.0 / s
            probs_ref[0, 0, h, :, :GS] = ea * r
            probs_ref[0, 0, h, :, GS:] = eb * r
            o1 = (jnp.dot(ea.astype(bf), va_ref[0][:, sl],
                          preferred_element_type=f32)
                  + jnp.dot(eb.astype(bf), vb_ref[0][:, sl],
                            preferred_element_type=f32)) * r[:, :DH]
            ge = jnp.exp2(gl * c).astype(bf)
            sg = jnp.dot(ge, ones, preferred_element_type=f32)
            o2 = jnp.dot(ge, vg_ref[h],
                         preferred_element_type=f32) / sg[:, :DH]
            acc_ref[:, sl] = o1 + o2

        prev = None
        for h in range(HEADS):
            cur = (h,) + head_logits(h)
            if prev is not None:
                head_post(*prev)
            prev = cur
        head_post(*prev)
        y_ref[0] = lax.dot_general(acc_ref[...].astype(bf), wp_ref[...],
                                   dn, preferred_element_type=f32)

    return pl.pallas_call(
        body,
        grid=(NB, NG),
        in_specs=[
            pl.BlockSpec((1, GS, DIM), lambda b, g: (b, g, 0)),
            pl.BlockSpec((1, GS, DIM), lambda b, g: (b, g, 0)),
            pl.BlockSpec((1, GS, DIM), lambda b, g: (b, g + 1, 0)),
            pl.BlockSpec((1, GS, DIM), lambda b, g: (b, g, 0)),
            pl.BlockSpec((1, GS, DIM), lambda b, g: (b, g + 1, 0)),
            pl.BlockSpec((HEADS, MG, DH), lambda b, g: (0, 0, 0)),
            pl.BlockSpec((HEADS, MG, DH), lambda b, g: (0, 0, 0)),
            pl.BlockSpec((DIM, DIM), lambda b, g: (0, 0)),
            pl.BlockSpec((GS, GS), lambda b, g: (0, 0)),
        ],
        out_specs=[
            pl.BlockSpec((1, 1, HEADS, GS, WIN), lambda b, g: (b, g, 0, 0, 0)),
            pl.BlockSpec((1, GS, DIM), lambda b, g: (b, g, 0)),
        ],
        out_shape=[
            jax.ShapeDtypeStruct((NB, NG, HEADS, GS, WIN), jnp.float32),
            jax.ShapeDtypeStruct((NB, N, DIM), jnp.float32),
        ],
        scratch_shapes=[pltpu.VMEM((GS, DIM), jnp.float32)],
    )(qbuf, kbuf, kbuf, vbuf, vbuf, k_global, v_global, wp,
      jnp.ones((GS, GS), jnp.bfloat16))


def kernel(normed_x, idx_last, k_global, v_global, Wq, Wk, Wv, Wproj):
    b, n, d = normed_x.shape
    idx = idx_last[..., 0].astype(jnp.int32)
    idx_ext = jnp.concatenate([idx, idx[:, n - GS:][:, ::-1]], axis=1)
    boff = (jnp.arange(b, dtype=jnp.int32) * n)[:, None]
    flat_idx = (idx_ext + boff).reshape(-1)
    xg = _sc_gather(normed_x.reshape(b * n, d), flat_idx, chunk=24)
    xg = xg.reshape(b, n + GS, d)

    bf = jnp.bfloat16
    qbuf, kbuf, vbuf = _proj_qkv(xg, Wq.astype(bf), Wk.astype(bf),
                                 Wv.astype(bf))
    probs, y = _attn(qbuf, kbuf, vbuf, k_global.astype(bf),
                     v_global.astype(bf), Wproj.astype(bf))

    idx_d = idx + (xg[0, 0, 0] * 0.0).astype(jnp.int32)
    ar = jnp.arange(n, dtype=jnp.int32)
    maxsrc = jax.vmap(
        lambda i: jnp.full((n,), -1, jnp.int32).at[i].max(ar))(idx_d)
    src = jnp.where(maxsrc >= 0, maxsrc, ar[None])
    src_flat = (src + boff).reshape(-1)
    out = _sc_gather(y.reshape(b * n, d), src_flat, chunk=32).reshape(b, n, d)
    return out, probs


# speedup vs baseline: 1.1963x; 1.1963x over previous
import functools

import jax
import jax.numpy as jnp
from jax import lax
from jax.experimental import pallas as pl
from jax.experimental.pallas import tpu as pltpu
from jax.experimental.pallas import tpu_sc as plsc

DIM = 1024
HEADS = 16
DH = 64
GS = 128
WIN = 2 * GS
NB = 2
N = 4096
NG = N // GS
MG = 128
SCALE = DH ** -0.5


def _sc_gather(table, idx, chunk):
    R, D = table.shape
    (M,) = idx.shape
    info = plsc.get_sparse_core_info()
    nw = info.num_cores * info.num_subcores
    per_w = M // nw
    assert per_w * nw == M and per_w % chunk == 0 and chunk % 8 == 0
    nchunks = per_w // chunk
    mesh = plsc.VectorSubcoreMesh(core_axis_name="c", subcore_axis_name="s")

    @functools.partial(
        pl.kernel,
        mesh=mesh,
        out_type=jax.ShapeDtypeStruct((M, D), table.dtype),
        scratch_types=[
            pltpu.VMEM((per_w,), jnp.int32),
            pltpu.VMEM((chunk, D), table.dtype),
            pltpu.VMEM((chunk, D), table.dtype),
            pltpu.SemaphoreType.DMA,
            pltpu.SemaphoreType.DMA,
        ],
    )
    def gk(table_hbm, idx_hbm, out_hbm, idx_v, rows_a, rows_b, sem_a, sem_b):
        wid = lax.axis_index("s") * info.num_cores + lax.axis_index("c")
        base = wid * per_w
        pltpu.sync_copy(idx_hbm.at[pl.ds(base, per_w)], idx_v)
        bufs = (rows_a, rows_b)
        sems = (sem_a, sem_b)
        copies = [
            pltpu.async_copy(
                table_hbm.at[idx_v.at[pl.ds(c * chunk, chunk)]],
                bufs[c % 2],
                sems[c % 2],
            )
            for c in range(min(2, nchunks))
        ]
        for c in range(nchunks):
            copies[c].wait()
            pltpu.sync_copy(bufs[c % 2], out_hbm.at[pl.ds(base + c * chunk, chunk)])
            if c + 2 < nchunks:
                copies.append(
                    pltpu.async_copy(
                        table_hbm.at[idx_v.at[pl.ds((c + 2) * chunk, chunk)]],
                        bufs[c % 2],
                        sems[c % 2],
                    )
                )

    return gk(table, idx)


def _proj_kv(xg, wk_t, wv_t):
    nb = xg.shape[1] // GS

    def body(x_ref, wk_ref, wv_ref, k_ref, v_ref):
        x = x_ref[0].astype(jnp.bfloat16)
        k_ref[0] = jnp.dot(x, wk_ref[...], preferred_element_type=jnp.float32
                           ).astype(jnp.bfloat16)
        v_ref[0] = jnp.dot(x, wv_ref[...], preferred_element_type=jnp.float32
                           ).astype(jnp.bfloat16)

    return pl.pallas_call(
        body,
        grid=(NB, nb),
        in_specs=[
            pl.BlockSpec((1, GS, DIM), lambda b, g: (b, g, 0)),
            pl.BlockSpec((DIM, DIM), lambda b, g: (0, 0)),
            pl.BlockSpec((DIM, DIM), lambda b, g: (0, 0)),
        ],
        out_specs=[
            pl.BlockSpec((1, GS, DIM), lambda b, g: (b, g, 0)),
            pl.BlockSpec((1, GS, DIM), lambda b, g: (b, g, 0)),
        ],
        out_shape=[
            jax.ShapeDtypeStruct((NB, nb * GS, DIM), jnp.bfloat16),
            jax.ShapeDtypeStruct((NB, nb * GS, DIM), jnp.bfloat16),
        ],
    )(xg, wk_t, wv_t)


def _attn(xg, kbuf, vbuf, k_global, v_global, wq_t, wp_t):
    c = SCALE * 1.4426950408889634
    bf = jnp.bfloat16
    f32 = jnp.float32

    def body(xq_ref, ka_ref, kb_ref, va_ref, vb_ref, kg_ref, vg_ref,
             wq_ref, wp_ref, ones_ref, probs_ref, y_ref, acc_ref):
        q = jnp.dot(xq_ref[0].astype(bf), wq_ref[...],
                    preferred_element_type=f32).astype(bf)
        ones = ones_ref[...]
        dn = (((1,), (1,)), ((), ()))

        def head_logits(h):
            sl = slice(h * DH, (h + 1) * DH)
            qh = q[:, sl]
            la = lax.dot_general(qh, ka_ref[0][:, sl], dn,
                                 preferred_element_type=f32)
            lb = lax.dot_general(qh, kb_ref[0][:, sl], dn,
                                 preferred_element_type=f32)
            gl = lax.dot_general(qh, kg_ref[h], dn,
                                 preferred_element_type=f32)
            return la, lb, gl

        def head_post(h, la, lb, gl):
            sl = slice(h * DH, (h + 1) * DH)
            ea = jnp.exp2(la * c)
            eb = jnp.exp2(lb * c)
            eab = (ea + eb).astype(bf)
            s = jnp.dot(eab, ones, preferred_element_type=f32)
            r = 1.0 / s
            probs_ref[0, 0, h, :, :GS] = ea * r
            probs_ref[0, 0, h, :, GS:] = eb * r
            o1 = (jnp.dot(ea.astype(bf), va_ref[0][:, sl],
                          preferred_element_type=f32)
                  + jnp.dot(eb.astype(bf), vb_ref[0][:, sl],
                            preferred_element_type=f32)) * r[:, :DH]
            ge = jnp.exp2(gl * c).astype(bf)
            sg = jnp.dot(ge, ones, preferred_element_type=f32)
            o2 = jnp.dot(ge, vg_ref[h],
                         preferred_element_type=f32) / sg[:, :DH]
            acc_ref[:, sl] = o1 + o2

        prev = None
        for h in range(HEADS):
            cur = (h,) + head_logits(h)
            if prev is not None:
                head_post(*prev)
            prev = cur
        head_post(*prev)
        y_ref[0] = jnp.dot(acc_ref[...].astype(bf), wp_ref[...],
                           preferred_element_type=f32)

    return pl.pallas_call(
        body,
        grid=(NB, NG),
        in_specs=[
            pl.BlockSpec((1, GS, DIM), lambda b, g: (b, g, 0)),
            pl.BlockSpec((1, GS, DIM), lambda b, g: (b, g, 0)),
            pl.BlockSpec((1, GS, DIM), lambda b, g: (b, g + 1, 0)),
            pl.BlockSpec((1, GS, DIM), lambda b, g: (b, g, 0)),
            pl.BlockSpec((1, GS, DIM), lambda b, g: (b, g + 1, 0)),
            pl.BlockSpec((HEADS, MG, DH), lambda b, g: (0, 0, 0)),
            pl.BlockSpec((HEADS, MG, DH), lambda b, g: (0, 0, 0)),
            pl.BlockSpec((DIM, DIM), lambda b, g: (0, 0)),
            pl.BlockSpec((DIM, DIM), lambda b, g: (0, 0)),
            pl.BlockSpec((GS, GS), lambda b, g: (0, 0)),
        ],
        out_specs=[
            pl.BlockSpec((1, 1, HEADS, GS, WIN), lambda b, g: (b, g, 0, 0, 0)),
            pl.BlockSpec((1, GS, DIM), lambda b, g: (b, g, 0)),
        ],
        out_shape=[
            jax.ShapeDtypeStruct((NB, NG, HEADS, GS, WIN), jnp.float32),
            jax.ShapeDtypeStruct((NB, N, DIM), jnp.float32),
        ],
        scratch_shapes=[pltpu.VMEM((GS, DIM), jnp.float32)],
    )(xg, kbuf, kbuf, vbuf, vbuf, k_global, v_global, wq_t, wp_t,
      jnp.ones((GS, GS), jnp.bfloat16))


def kernel(normed_x, idx_last, k_global, v_global, Wq, Wk, Wv, Wproj):
    b, n, d = normed_x.shape
    idx = idx_last[..., 0].astype(jnp.int32)
    idx_ext = jnp.concatenate([idx, idx[:, n - GS:][:, ::-1]], axis=1)
    boff = (jnp.arange(b, dtype=jnp.int32) * n)[:, None]
    flat_idx = (idx_ext + boff).reshape(-1)
    xg = _sc_gather(normed_x.reshape(b * n, d), flat_idx, chunk=24)
    xg = xg.reshape(b, n + GS, d)

    bf = jnp.bfloat16
    kbuf, vbuf = _proj_kv(xg, Wk.T.astype(bf), Wv.T.astype(bf))
    probs, y = _attn(xg, kbuf, vbuf, k_global.astype(bf), v_global.astype(bf),
                     Wq.T.astype(bf), Wproj.T.astype(bf))

    idx_d = idx + (xg[0, 0, 0] * 0.0).astype(jnp.int32)
    ar = jnp.arange(n, dtype=jnp.int32)
    maxsrc = jax.vmap(
        lambda i: jnp.full((n,), -1, jnp.int32).at[i].max(ar))(idx_d)
    src = jnp.where(maxsrc >= 0, maxsrc, ar[None])
    src_flat = (src + boff).reshape(-1)
    out = _sc_gather(y.reshape(b * n, d), src_flat, chunk=32).reshape(b, n, d)
    return out, probs


# speedup vs baseline: 1.2491x; 1.0441x over previous
import functools

import jax
import jax.numpy as jnp
from jax import lax
from jax.experimental import pallas as pl
from jax.experimental.pallas import tpu as pltpu
from jax.experimental.pallas import tpu_sc as plsc

DIM = 1024
HEADS = 16
DH = 64
GS = 128
WIN = 2 * GS
NB = 2
N = 4096
NG = N // GS
MG = 128
SCALE = DH ** -0.5


def _sc_gather(table, idx, chunk):
    R, D = table.shape
    (M,) = idx.shape
    info = plsc.get_sparse_core_info()
    nw = info.num_cores * info.num_subcores
    per_w = M // nw
    assert per_w * nw == M and per_w % chunk == 0 and chunk % 8 == 0
    nchunks = per_w // chunk
    mesh = plsc.VectorSubcoreMesh(core_axis_name="c", subcore_axis_name="s")

    @functools.partial(
        pl.kernel,
        mesh=mesh,
        out_type=jax.ShapeDtypeStruct((M, D), table.dtype),
        scratch_types=[
            pltpu.VMEM((per_w,), jnp.int32),
            pltpu.VMEM((chunk, D), table.dtype),
            pltpu.VMEM((chunk, D), table.dtype),
            pltpu.SemaphoreType.DMA,
            pltpu.SemaphoreType.DMA,
        ],
    )
    def gk(table_hbm, idx_hbm, out_hbm, idx_v, rows_a, rows_b, sem_a, sem_b):
        wid = lax.axis_index("s") * info.num_cores + lax.axis_index("c")
        base = wid * per_w
        pltpu.sync_copy(idx_hbm.at[pl.ds(base, per_w)], idx_v)
        bufs = (rows_a, rows_b)
        sems = (sem_a, sem_b)
        copies = [
            pltpu.async_copy(
                table_hbm.at[idx_v.at[pl.ds(c * chunk, chunk)]],
                bufs[c % 2],
                sems[c % 2],
            )
            for c in range(min(2, nchunks))
        ]
        for c in range(nchunks):
            copies[c].wait()
            pltpu.sync_copy(bufs[c % 2], out_hbm.at[pl.ds(base + c * chunk, chunk)])
            if c + 2 < nchunks:
                copies.append(
                    pltpu.async_copy(
                        table_hbm.at[idx_v.at[pl.ds((c + 2) * chunk, chunk)]],
                        bufs[c % 2],
                        sems[c % 2],
                    )
                )

    return gk(table, idx)


def _proj_kv(xg, wk_t, wv_t):
    nb = xg.shape[1] // GS

    def body(x_ref, wk_ref, wv_ref, k_ref, v_ref):
        x = x_ref[0].astype(jnp.bfloat16)
        k_ref[0] = jnp.dot(x, wk_ref[...], preferred_element_type=jnp.float32
                           ).astype(jnp.bfloat16)
        v_ref[0] = jnp.dot(x, wv_ref[...], preferred_element_type=jnp.float32
                           ).astype(jnp.bfloat16)

    return pl.pallas_call(
        body,
        grid=(NB, nb),
        in_specs=[
            pl.BlockSpec((1, GS, DIM), lambda b, g: (b, g, 0)),
            pl.BlockSpec((DIM, DIM), lambda b, g: (0, 0)),
            pl.BlockSpec((DIM, DIM), lambda b, g: (0, 0)),
        ],
        out_specs=[
            pl.BlockSpec((1, GS, DIM), lambda b, g: (b, g, 0)),
            pl.BlockSpec((1, GS, DIM), lambda b, g: (b, g, 0)),
        ],
        out_shape=[
            jax.ShapeDtypeStruct((NB, nb * GS, DIM), jnp.bfloat16),
            jax.ShapeDtypeStruct((NB, nb * GS, DIM), jnp.bfloat16),
        ],
    )(xg, wk_t, wv_t)


def _attn(xg, kbuf, vbuf, k_global, v_global, wq_t, wp_t):
    c = SCALE * 1.4426950408889634
    bf = jnp.bfloat16
    f32 = jnp.float32

    def body(xq_ref, ka_ref, kb_ref, va_ref, vb_ref, kg_ref, vg_ref,
             wq_ref, wp_ref, ones_ref, probs_ref, y_ref, acc_ref):
        q = jnp.dot(xq_ref[0].astype(bf), wq_ref[...],
                    preferred_element_type=f32).astype(bf)
        ones = ones_ref[...]
        dn = (((1,), (1,)), ((), ()))

        def head_logits(h):
            sl = slice(h * DH, (h + 1) * DH)
            qh = q[:, sl]
            la = lax.dot_general(qh, ka_ref[0][:, sl], dn,
                                 preferred_element_type=f32)
            lb = lax.dot_general(qh, kb_ref[0][:, sl], dn,
                                 preferred_element_type=f32)
            gl = lax.dot_general(qh, kg_ref[h], dn,
                                 preferred_element_type=f32)
            return la, lb, gl

        def head_post(h, la, lb, gl):
            sl = slice(h * DH, (h + 1) * DH)
            ea = jnp.exp2(la * c)
            eb = jnp.exp2(lb * c)
            eab = (ea + eb).astype(bf)
            s = jnp.dot(eab, ones, preferred_element_type=f32)
            r = 1.0 / s
            probs_ref[0, 0, h, :, :GS] = ea * r
            probs_ref[0, 0, h, :, GS:] = eb * r
            o1 = (jnp.dot(ea.astype(bf), va_ref[0][:, sl],
                          preferred_element_type=f32)
                  + jnp.dot(eb.astype(bf), vb_ref[0][:, sl],
                            preferred_element_type=f32)) * r[:, :DH]
            ge = jnp.exp2(gl * c).astype(bf)
            sg = jnp.dot(ge, ones, preferred_element_type=f32)
            o2 = jnp.dot(ge, vg_ref[h],
                         preferred_element_type=f32) / sg[:, :DH]
            acc_ref[:, sl] = o1 + o2

        pipe = []
        for h in range(HEADS):
            pipe.append((h,) + head_logits(h))
            if len(pipe) > 3:
                head_post(*pipe.pop(0))
        for st in pipe:
            head_post(*st)
        y_ref[0] = jnp.dot(acc_ref[...].astype(bf), wp_ref[...],
                           preferred_element_type=f32)

    return pl.pallas_call(
        body,
        grid=(NB, NG),
        in_specs=[
            pl.BlockSpec((1, GS, DIM), lambda b, g: (b, g, 0)),
            pl.BlockSpec((1, GS, DIM), lambda b, g: (b, g, 0)),
            pl.BlockSpec((1, GS, DIM), lambda b, g: (b, g + 1, 0)),
            pl.BlockSpec((1, GS, DIM), lambda b, g: (b, g, 0)),
            pl.BlockSpec((1, GS, DIM), lambda b, g: (b, g + 1, 0)),
            pl.BlockSpec((HEADS, MG, DH), lambda b, g: (0, 0, 0)),
            pl.BlockSpec((HEADS, MG, DH), lambda b, g: (0, 0, 0)),
            pl.BlockSpec((DIM, DIM), lambda b, g: (0, 0)),
            pl.BlockSpec((DIM, DIM), lambda b, g: (0, 0)),
            pl.BlockSpec((GS, GS), lambda b, g: (0, 0)),
        ],
        out_specs=[
            pl.BlockSpec((1, 1, HEADS, GS, WIN), lambda b, g: (b, g, 0, 0, 0)),
            pl.BlockSpec((1, GS, DIM), lambda b, g: (b, g, 0)),
        ],
        out_shape=[
            jax.ShapeDtypeStruct((NB, NG, HEADS, GS, WIN), jnp.float32),
            jax.ShapeDtypeStruct((NB, N, DIM), jnp.float32),
        ],
        scratch_shapes=[pltpu.VMEM((GS, DIM), jnp.float32)],
    )(xg, kbuf, kbuf, vbuf, vbuf, k_global, v_global, wq_t, wp_t,
      jnp.ones((GS, GS), jnp.bfloat16))


def kernel(normed_x, idx_last, k_global, v_global, Wq, Wk, Wv, Wproj):
    b, n, d = normed_x.shape
    idx = idx_last[..., 0].astype(jnp.int32)
    idx_ext = jnp.concatenate([idx, idx[:, n - GS:][:, ::-1]], axis=1)
    boff = (jnp.arange(b, dtype=jnp.int32) * n)[:, None]
    flat_idx = (idx_ext + boff).reshape(-1)
    xg = _sc_gather(normed_x.reshape(b * n, d), flat_idx, chunk=24)
    xg = xg.reshape(b, n + GS, d)

    bf = jnp.bfloat16
    kbuf, vbuf = _proj_kv(xg, Wk.T.astype(bf), Wv.T.astype(bf))
    probs, y = _attn(xg, kbuf, vbuf, k_global.astype(bf), v_global.astype(bf),
                     Wq.T.astype(bf), Wproj.T.astype(bf))

    idx_d = idx + (xg[0, 0, 0] * 0.0).astype(jnp.int32)
    ar = jnp.arange(n, dtype=jnp.int32)
    maxsrc = jax.vmap(
        lambda i: jnp.full((n,), -1, jnp.int32).at[i].max(ar))(idx_d)
    src = jnp.where(maxsrc >= 0, maxsrc, ar[None])
    src_flat = (src + boff).reshape(-1)
    out = _sc_gather(y.reshape(b * n, d), src_flat, chunk=32).reshape(b, n, d)
    return out, probs
